# Initial kernel scaffold; baseline (speedup 1.0000x reference)
import functools

import jax
import jax.numpy as jnp
from jax import lax
from jax.experimental import pallas as pl
from jax.experimental.pallas import tpu as pltpu
from jax.experimental.pallas import tpu_sc as plsc

NUM_SEGMENTS = 512
NC = 2
NS = 16
L = 16
CR = 32

_NEG_INF = float("-inf")


@functools.lru_cache(maxsize=None)
def _make_seg_reduce(n, d, b):
    nw = NC * NS
    seg_per_w = b // nw
    n_sub = d // L
    mesh = plsc.VectorSubcoreMesh(core_axis_name="c", subcore_axis_name="s")

    @functools.partial(
        pl.kernel,
        out_type=(
            jax.ShapeDtypeStruct((b, d), jnp.float32),
            jax.ShapeDtypeStruct((b, d), jnp.float32),
        ),
        mesh=mesh,
        scratch_types=[
            pltpu.VMEM((n,), jnp.int32),
            pltpu.VMEM((CR, d), jnp.float32),
            pltpu.VMEM((seg_per_w, d), jnp.float32),
            pltpu.VMEM((seg_per_w, d), jnp.float32),
            pltpu.VMEM((L,), jnp.int32),
            pltpu.VMEM((L,), jnp.int32),
        ],
    )
    def seg_reduce(memb_hbm, x_hbm, max_hbm, sum_hbm,
                   memb_v, buf_v, omax_v, osum_v, lo_v, hi_v):
        wid = lax.axis_index("c") * NS + lax.axis_index("s")
        seg0 = wid * seg_per_w

        pltpu.sync_copy(memb_hbm, memb_v)

        targets = seg0 + lax.iota(jnp.int32, L)

        def lower_bound(tv):
            def step(_, carry):
                lo, hi = carry
                mid = lax.div(lo + hi, 2)
                vals = plsc.load_gather(memb_v, [mid])
                pred = vals < tv
                return jnp.where(pred, mid + 1, lo), jnp.where(pred, hi, mid)
            lo = jnp.zeros((L,), jnp.int32)
            hi = jnp.full((L,), n, jnp.int32)
            lo, hi = lax.fori_loop(0, 17, step, (lo, hi))
            return lo

        lo_v[...] = lower_bound(targets)
        hi_v[...] = lower_bound(targets + 1)

        def accum_rows(accs, valid_of_row):
            smax, ssum = list(accs[0]), list(accs[1])
            for r in range(CR):
                vmask = valid_of_row(r)
                for c in range(n_sub):
                    v = buf_v[r, pl.ds(c * L, L)]
                    if vmask is None:
                        vm = vs = v
                    else:
                        vm = jnp.where(vmask, v, _NEG_INF)
                        vs = jnp.where(vmask, v, 0.0)
                    smax[c] = jnp.maximum(smax[c], vm)
                    ssum[c] = ssum[c] + vs
            return tuple(smax), tuple(ssum)

        def do_segment(s_idx, _):
            lo = lo_v[s_idx]
            hi = hi_v[s_idx]
            nfull = lax.div(hi - lo, CR)

            zero16 = jnp.zeros((L,), jnp.float32)
            ninf16 = jnp.full((L,), _NEG_INF, jnp.float32)
            accs = (tuple(ninf16 for _ in range(n_sub)),
                    tuple(zero16 for _ in range(n_sub)))

            def full_chunk(k, accs):
                pltpu.sync_copy(x_hbm.at[pl.ds(lo + k * CR, CR)], buf_v)
                return accum_rows(accs, lambda r: None)

            accs = lax.fori_loop(0, nfull, full_chunk, accs)

            def tail(accs):
                base = jnp.maximum(hi - CR, 0)
                pltpu.sync_copy(x_hbm.at[pl.ds(base, CR)], buf_v)
                first_valid = lo + nfull * CR
                return accum_rows(
                    accs,
                    lambda r: jnp.logical_and(base + r >= first_valid,
                                              base + r < hi))

            accs = lax.cond(lo + nfull * CR < hi, tail, lambda a: a, accs)

            smax, ssum = accs
            for c in range(n_sub):
                m = smax[c]
                m = jnp.where(m == _NEG_INF, 0.0, m)
                omax_v[s_idx, pl.ds(c * L, L)] = m
                osum_v[s_idx, pl.ds(c * L, L)] = ssum[c]
            return 0

        lax.fori_loop(0, seg_per_w, do_segment, 0)

        pltpu.sync_copy(omax_v, max_hbm.at[pl.ds(seg0, seg_per_w)])
        pltpu.sync_copy(osum_v, sum_hbm.at[pl.ds(seg0, seg_per_w)])

    return seg_reduce


def _merge_body(mx_ref, sm_ref, wa_ref, wb_ref, b_ref, o_ref):
    acc = lax.dot_general(mx_ref[...], wa_ref[...], (((1,), (1,)), ((), ())),
                          preferred_element_type=jnp.float32)
    acc = acc + lax.dot_general(sm_ref[...], wb_ref[...],
                                (((1,), (1,)), ((), ())),
                                preferred_element_type=jnp.float32)
    o_ref[...] = acc + b_ref[...]


def kernel(x, membership, W_merge, b_merge):
    n, d = x.shape
    bseg = NUM_SEGMENTS
    memb32 = membership.astype(jnp.int32)
    seg_max, seg_sum = _make_seg_reduce(n, d, bseg)(memb32, x)
    wa = W_merge[:, :d]
    wb = W_merge[:, d:]
    out = pl.pallas_call(
        _merge_body,
        out_shape=jax.ShapeDtypeStruct((bseg, d), jnp.float32),
    )(seg_max, seg_sum, wa, wb, b_merge.reshape(1, d))
    return out


# speedup vs baseline: 3.6881x; 3.6881x over previous
import functools

import jax
import jax.numpy as jnp
from jax import lax
from jax.experimental import pallas as pl
from jax.experimental.pallas import tpu as pltpu
from jax.experimental.pallas import tpu_sc as plsc

NUM_SEGMENTS = 512
NC = 2
NS = 16
L = 16
CR = 32

_NEG_INF = float("-inf")


@functools.lru_cache(maxsize=None)
def _make_seg_reduce(n, d, b):
    nw = NC * NS
    seg_per_w = b // nw
    n_sub = d // L
    assert n % 8 == 0 and n >= CR and d % L == 0 and b % nw == 0
    mesh = plsc.VectorSubcoreMesh(core_axis_name="c", subcore_axis_name="s",
                                  num_cores=NC, num_subcores=NS)

    @functools.partial(
        pl.kernel,
        out_type=(
            jax.ShapeDtypeStruct((b, d), jnp.float32),
            jax.ShapeDtypeStruct((b, d), jnp.float32),
        ),
        mesh=mesh,
        scratch_types=[
            pltpu.VMEM((n,), jnp.int32),
            pltpu.VMEM((CR, d), jnp.float32),
            pltpu.VMEM((seg_per_w, d), jnp.float32),
            pltpu.VMEM((seg_per_w, d), jnp.float32),
        ],
        compiler_params=pltpu.CompilerParams(use_tc_tiling_on_sc=False,
                                             needs_layout_passes=False),
    )
    def seg_reduce(memb_hbm, x_hbm, max_hbm, sum_hbm,
                   memb_v, buf_v, omax_v, osum_v):
        wid = lax.axis_index("c") * NS + lax.axis_index("s")
        seg0 = wid * seg_per_w

        pltpu.sync_copy(memb_hbm, memb_v)

        targets = seg0 + lax.iota(jnp.int32, L)

        def lower_bound(tv):
            def step(_, carry):
                lo, hi = carry
                mid = lax.div(lo + hi, 2)
                vals = plsc.load_gather(memb_v, [jnp.minimum(mid, n - 1)])
                pred = vals < tv
                return jnp.where(pred, mid + 1, lo), jnp.where(pred, hi, mid)
            lo = jnp.zeros((L,), jnp.int32)
            hi = jnp.full((L,), n, jnp.int32)
            lo, hi = lax.fori_loop(0, 17, step, (lo, hi))
            return jnp.minimum(lo, n)

        starts = lower_bound(targets)
        ends = lower_bound(targets + 1)
        lanes = lax.iota(jnp.int32, L)

        def lane_extract(vec, idx):
            return jnp.sum(jnp.where(lanes == idx, vec, 0), axis=0)

        def accum_rows(accs, valid_of_row):
            smax, ssum = list(accs[0]), list(accs[1])
            for r in range(CR):
                vmask = valid_of_row(r)
                for c in range(n_sub):
                    v = buf_v[r, pl.ds(c * L, L)]
                    if vmask is None:
                        vm = vs = v
                    else:
                        vm = jnp.where(vmask, v, _NEG_INF)
                        vs = jnp.where(vmask, v, 0.0)
                    smax[c] = jnp.maximum(smax[c], vm)
                    ssum[c] = ssum[c] + vs
            return tuple(smax), tuple(ssum)

        def do_segment(s_idx, _):
            lo = lane_extract(starts, s_idx)
            hi = lane_extract(ends, s_idx)
            lo8u = lax.div(lo + 7, 8) * 8
            nfull = lax.div(jnp.maximum(hi - lo8u, 0), CR)

            zero16 = jnp.zeros((L,), jnp.float32)
            ninf16 = jnp.full((L,), _NEG_INF, jnp.float32)
            accs = (tuple(ninf16 for _ in range(n_sub)),
                    tuple(zero16 for _ in range(n_sub)))

            def masked_chunk(accs, base, vlo, vhi):
                pltpu.sync_copy(
                    x_hbm.at[pl.ds(pl.multiple_of(base, 8), CR)], buf_v)
                return accum_rows(
                    accs,
                    lambda r: jnp.logical_and(base + r >= vlo,
                                              base + r < vhi))

            def head(accs):
                base = jnp.minimum(jnp.maximum(lo8u - 8, 0), n - CR)
                return masked_chunk(accs, base, lo, jnp.minimum(lo8u, hi))

            accs = lax.cond(lo < jnp.minimum(lo8u, hi), head,
                            lambda a: a, accs)

            def full_chunk(k, accs):
                base = pl.multiple_of(lo8u + k * CR, 8)
                pltpu.sync_copy(x_hbm.at[pl.ds(base, CR)], buf_v)
                return accum_rows(accs, lambda r: None)

            accs = lax.fori_loop(0, nfull, full_chunk, accs)

            t_lo = lo8u + nfull * CR

            def tail(accs):
                base = jnp.minimum(t_lo, n - CR)
                return masked_chunk(accs, base, t_lo, hi)

            accs = lax.cond(t_lo < hi, tail, lambda a: a, accs)

            smax, ssum = accs
            for c in range(n_sub):
                m = smax[c]
                m = jnp.where(m == _NEG_INF, 0.0, m)
                omax_v[s_idx, pl.ds(c * L, L)] = m
                osum_v[s_idx, pl.ds(c * L, L)] = ssum[c]
            return 0

        lax.fori_loop(0, seg_per_w, do_segment, 0)

        pltpu.sync_copy(omax_v, max_hbm.at[pl.ds(seg0, seg_per_w)])
        pltpu.sync_copy(osum_v, sum_hbm.at[pl.ds(seg0, seg_per_w)])

    return seg_reduce


def _merge_body(mx_ref, sm_ref, wa_ref, wb_ref, b_ref, o_ref):
    acc = lax.dot_general(mx_ref[...], wa_ref[...], (((1,), (1,)), ((), ())),
                          preferred_element_type=jnp.float32)
    acc = acc + lax.dot_general(sm_ref[...], wb_ref[...],
                                (((1,), (1,)), ((), ())),
                                preferred_element_type=jnp.float32)
    o_ref[...] = acc + b_ref[...]


def kernel(x, membership, W_merge, b_merge):
    n, d = x.shape
    bseg = NUM_SEGMENTS
    memb32 = membership.astype(jnp.int32)
    seg_max, seg_sum = _make_seg_reduce(n, d, bseg)(memb32, x)
    wa = W_merge[:, :d]
    wb = W_merge[:, d:]
    out = pl.pallas_call(
        _merge_body,
        out_shape=jax.ShapeDtypeStruct((bseg, d), jnp.float32),
    )(seg_max, seg_sum, wa, wb, b_merge.reshape(1, d))
    return out


# speedup vs baseline: 4.5517x; 1.2342x over previous
import functools

import jax
import jax.numpy as jnp
from jax import lax
from jax.experimental import pallas as pl
from jax.experimental.pallas import tpu as pltpu
from jax.experimental.pallas import tpu_sc as plsc

NUM_SEGMENTS = 512
NC = 2
NS = 16
L = 16
CR = 32

_NEG_INF = float("-inf")


@functools.lru_cache(maxsize=None)
def _make_seg_reduce(n, d, b):
    nw = NC * NS
    seg_per_w = b // nw
    n_sub = d // L
    assert n % 8 == 0 and n >= CR and d % L == 0 and b % nw == 0
    mesh = plsc.VectorSubcoreMesh(core_axis_name="c", subcore_axis_name="s",
                                  num_cores=NC, num_subcores=NS)

    @functools.partial(
        pl.kernel,
        out_type=(
            jax.ShapeDtypeStruct((b, d), jnp.float32),
            jax.ShapeDtypeStruct((b, d), jnp.float32),
        ),
        mesh=mesh,
        scratch_types=[
            pltpu.VMEM((n,), jnp.int32),
            pltpu.VMEM((CR, d), jnp.float32),
            pltpu.VMEM((seg_per_w, d), jnp.float32),
            pltpu.VMEM((seg_per_w, d), jnp.float32),
        ],
        compiler_params=pltpu.CompilerParams(needs_layout_passes=False),
    )
    def seg_reduce(memb_hbm, x_hbm, max_hbm, sum_hbm,
                   memb_v, buf_v, omax_v, osum_v):
        wid = lax.axis_index("c") * NS + lax.axis_index("s")
        seg0 = wid * seg_per_w

        pltpu.sync_copy(memb_hbm, memb_v)

        targets = seg0 + lax.iota(jnp.int32, L)

        def lower_bound(tv):
            def step(_, carry):
                lo, hi = carry
                mid = lax.div(lo + hi, 2)
                vals = plsc.load_gather(memb_v, [jnp.minimum(mid, n - 1)])
                pred = vals < tv
                return jnp.where(pred, mid + 1, lo), jnp.where(pred, hi, mid)
            lo = jnp.zeros((L,), jnp.int32)
            hi = jnp.full((L,), n, jnp.int32)
            lo, hi = lax.fori_loop(0, 17, step, (lo, hi))
            return jnp.minimum(lo, n)

        starts = lower_bound(targets)
        ends = lower_bound(targets + 1)
        lanes = lax.iota(jnp.int32, L)

        def lane_extract(vec, idx):
            return jnp.sum(jnp.where(lanes == idx, vec, 0), axis=0)

        def accum_rows(accs, valid_of_row):
            smax, ssum = list(accs[0]), list(accs[1])
            for r in range(CR):
                vmask = valid_of_row(r)
                for c in range(n_sub):
                    v = buf_v[r, pl.ds(c * L, L)]
                    if vmask is None:
                        vm = vs = v
                    else:
                        vm = jnp.where(vmask, v, _NEG_INF)
                        vs = jnp.where(vmask, v, 0.0)
                    smax[c] = jnp.maximum(smax[c], vm)
                    ssum[c] = ssum[c] + vs
            return tuple(smax), tuple(ssum)

        def do_segment(s_idx, _):
            lo = lane_extract(starts, s_idx)
            hi = lane_extract(ends, s_idx)
            lo8u = lax.div(lo + 7, 8) * 8
            nfull = lax.div(jnp.maximum(hi - lo8u, 0), CR)

            zero16 = jnp.zeros((L,), jnp.float32)
            ninf16 = jnp.full((L,), _NEG_INF, jnp.float32)
            accs = (tuple(ninf16 for _ in range(n_sub)),
                    tuple(zero16 for _ in range(n_sub)))

            def masked_chunk(accs, base, vlo, vhi):
                pltpu.sync_copy(
                    x_hbm.at[pl.ds(pl.multiple_of(base, 8), CR)], buf_v)
                return accum_rows(
                    accs,
                    lambda r: jnp.logical_and(base + r >= vlo,
                                              base + r < vhi))

            def head(accs):
                base = jnp.minimum(jnp.maximum(lo8u - 8, 0), n - CR)
                return masked_chunk(accs, base, lo, jnp.minimum(lo8u, hi))

            accs = lax.cond(lo < jnp.minimum(lo8u, hi), head,
                            lambda a: a, accs)

            def full_chunk(k, accs):
                base = pl.multiple_of(lo8u + k * CR, 8)
                pltpu.sync_copy(x_hbm.at[pl.ds(base, CR)], buf_v)
                return accum_rows(accs, lambda r: None)

            accs = lax.fori_loop(0, nfull, full_chunk, accs)

            t_lo = lo8u + nfull * CR

            def tail(accs):
                base = jnp.minimum(t_lo, n - CR)
                return masked_chunk(accs, base, t_lo, hi)

            accs = lax.cond(t_lo < hi, tail, lambda a: a, accs)

            smax, ssum = accs
            for c in range(n_sub):
                m = smax[c]
                m = jnp.where(m == _NEG_INF, 0.0, m)
                omax_v[s_idx, pl.ds(c * L, L)] = m
                osum_v[s_idx, pl.ds(c * L, L)] = ssum[c]
            return 0

        lax.fori_loop(0, seg_per_w, do_segment, 0)

        pltpu.sync_copy(omax_v, max_hbm.at[pl.ds(seg0, seg_per_w)])
        pltpu.sync_copy(osum_v, sum_hbm.at[pl.ds(seg0, seg_per_w)])

    return seg_reduce


def _merge_body(mx_ref, sm_ref, wa_ref, wb_ref, b_ref, o_ref):
    acc = lax.dot_general(mx_ref[...], wa_ref[...], (((1,), (1,)), ((), ())),
                          preferred_element_type=jnp.float32)
    acc = acc + lax.dot_general(sm_ref[...], wb_ref[...],
                                (((1,), (1,)), ((), ())),
                                preferred_element_type=jnp.float32)
    o_ref[...] = acc + b_ref[...]


def kernel(x, membership, W_merge, b_merge):
    n, d = x.shape
    bseg = NUM_SEGMENTS
    memb32 = membership.astype(jnp.int32)
    seg_max, seg_sum = _make_seg_reduce(n, d, bseg)(memb32, x)
    wa = W_merge[:, :d]
    wb = W_merge[:, d:]
    out = pl.pallas_call(
        _merge_body,
        out_shape=jax.ShapeDtypeStruct((bseg, d), jnp.float32),
    )(seg_max, seg_sum, wa, wb, b_merge.reshape(1, d))
    return out


# speedup vs baseline: 4.5908x; 1.0086x over previous
import functools

import jax
import jax.numpy as jnp
from jax import lax
from jax.experimental import pallas as pl
from jax.experimental.pallas import tpu as pltpu
from jax.experimental.pallas import tpu_sc as plsc

NUM_SEGMENTS = 512
NC = 2
NS = 16
L = 16
CR = 32

_NEG_INF = float("-inf")


@functools.lru_cache(maxsize=None)
def _make_seg_reduce(n, d, b):
    nw = NC * NS
    seg_per_w = b // nw
    n_sub = d // L
    assert n % 8 == 0 and n >= CR and d % L == 0 and b % nw == 0
    mesh = plsc.VectorSubcoreMesh(core_axis_name="c", subcore_axis_name="s",
                                  num_cores=NC, num_subcores=NS)

    @functools.partial(
        pl.kernel,
        out_type=(
            jax.ShapeDtypeStruct((b, d), jnp.float32),
            jax.ShapeDtypeStruct((b, d), jnp.float32),
        ),
        mesh=mesh,
        scratch_types=[
            pltpu.VMEM((n,), jnp.int32),
            pltpu.VMEM((8, d), jnp.float32),
            pltpu.VMEM((CR, d), jnp.float32),
            pltpu.VMEM((CR, d), jnp.float32),
            pltpu.VMEM((CR, d), jnp.float32),
            pltpu.VMEM((seg_per_w, d), jnp.float32),
            pltpu.VMEM((seg_per_w, d), jnp.float32),
            pltpu.SemaphoreType.DMA,
            pltpu.SemaphoreType.DMA,
            pltpu.SemaphoreType.DMA,
            pltpu.SemaphoreType.DMA,
        ],
        compiler_params=pltpu.CompilerParams(needs_layout_passes=False),
    )
    def seg_reduce(memb_hbm, x_hbm, max_hbm, sum_hbm,
                   memb_v, hbuf, fbuf0, fbuf1, tbuf, omax_v, osum_v,
                   sem_h, sem_f0, sem_f1, sem_t):
        wid = lax.axis_index("c") * NS + lax.axis_index("s")
        seg0 = wid * seg_per_w

        pltpu.sync_copy(memb_hbm, memb_v)

        targets = seg0 + lax.iota(jnp.int32, L)

        def lower_bound(tv):
            def step(_, carry):
                lo, hi = carry
                mid = lax.div(lo + hi, 2)
                vals = plsc.load_gather(memb_v, [jnp.minimum(mid, n - 1)])
                pred = vals < tv
                return jnp.where(pred, mid + 1, lo), jnp.where(pred, hi, mid)
            lo = jnp.zeros((L,), jnp.int32)
            hi = jnp.full((L,), n, jnp.int32)
            lo, hi = lax.fori_loop(0, 17, step, (lo, hi))
            return jnp.minimum(lo, n)

        starts = lower_bound(targets)
        ends = lower_bound(targets + 1)
        lanes = lax.iota(jnp.int32, L)

        def lane_extract(vec, idx):
            return jnp.sum(jnp.where(lanes == idx, vec, 0), axis=0)

        def accum_rows(buf, nrows, accs, valid_of_row):
            smax, ssum = list(accs[0]), list(accs[1])
            for r in range(nrows):
                vmask = valid_of_row(r)
                for c in range(n_sub):
                    v = buf[r, pl.ds(c * L, L)]
                    if vmask is None:
                        vm = vs = v
                    else:
                        vm = jnp.where(vmask, v, _NEG_INF)
                        vs = jnp.where(vmask, v, 0.0)
                    smax[c] = jnp.maximum(smax[c], vm)
                    ssum[c] = ssum[c] + vs
            return tuple(smax), tuple(ssum)

        def do_segment(s_idx, _):
            lo = lane_extract(starts, s_idx)
            hi = lane_extract(ends, s_idx)
            lo8u = lax.div(lo + 7, 8) * 8
            nfull = lax.div(jnp.maximum(hi - lo8u, 0), CR)
            head_hi = jnp.minimum(lo8u, hi)
            head_needed = lo < head_hi
            hbase = jnp.minimum(jnp.maximum(lo8u - 8, 0), n - 8)
            t_lo = lo8u + nfull * CR
            tail_needed = t_lo < hi
            tbase = jnp.minimum(t_lo, n - CR)

            def start_chunk(k, buf, sem):
                base = pl.multiple_of(lo8u + k * CR, 8)
                pltpu.make_async_copy(
                    x_hbm.at[pl.ds(base, CR)], buf, sem).start()

            @pl.when(nfull > 0)
            def _():
                start_chunk(0, fbuf0, sem_f0)

            @pl.when(head_needed)
            def _():
                base = pl.multiple_of(hbase, 8)
                pltpu.make_async_copy(
                    x_hbm.at[pl.ds(base, 8)], hbuf, sem_h).start()

            @pl.when(tail_needed)
            def _():
                base = pl.multiple_of(tbase, 8)
                pltpu.make_async_copy(
                    x_hbm.at[pl.ds(base, CR)], tbuf, sem_t).start()

            zero16 = jnp.zeros((L,), jnp.float32)
            ninf16 = jnp.full((L,), _NEG_INF, jnp.float32)
            accs = (tuple(ninf16 for _ in range(n_sub)),
                    tuple(zero16 for _ in range(n_sub)))

            def wait(buf, sem, rows):
                pltpu.make_async_copy(
                    x_hbm.at[pl.ds(0, rows)], buf, sem).wait()

            npairs = lax.div(nfull + 1, 2)

            def pair(i, accs):
                k0 = 2 * i
                wait(fbuf0, sem_f0, CR)

                @pl.when(k0 + 1 < nfull)
                def _():
                    start_chunk(k0 + 1, fbuf1, sem_f1)

                accs = accum_rows(fbuf0, CR, accs, lambda r: None)

                def odd(accs):
                    wait(fbuf1, sem_f1, CR)

                    @pl.when(k0 + 2 < nfull)
                    def _():
                        start_chunk(k0 + 2, fbuf0, sem_f0)

                    return accum_rows(fbuf1, CR, accs, lambda r: None)

                return lax.cond(k0 + 1 < nfull, odd, lambda a: a, accs)

            accs = lax.fori_loop(0, npairs, pair, accs)

            def head_fn(accs):
                wait(hbuf, sem_h, 8)
                return accum_rows(
                    hbuf, 8, accs,
                    lambda r: jnp.logical_and(hbase + r >= lo,
                                              hbase + r < head_hi))

            accs = lax.cond(head_needed, head_fn, lambda a: a, accs)

            def tail_fn(accs):
                wait(tbuf, sem_t, CR)
                return accum_rows(
                    tbuf, CR, accs,
                    lambda r: jnp.logical_and(tbase + r >= t_lo,
                                              tbase + r < hi))

            accs = lax.cond(tail_needed, tail_fn, lambda a: a, accs)

            smax, ssum = accs
            for c in range(n_sub):
                m = smax[c]
                m = jnp.where(m == _NEG_INF, 0.0, m)
                omax_v[s_idx, pl.ds(c * L, L)] = m
                osum_v[s_idx, pl.ds(c * L, L)] = ssum[c]
            return 0

        lax.fori_loop(0, seg_per_w, do_segment, 0)

        pltpu.sync_copy(omax_v, max_hbm.at[pl.ds(seg0, seg_per_w)])
        pltpu.sync_copy(osum_v, sum_hbm.at[pl.ds(seg0, seg_per_w)])

    return seg_reduce


def _merge_body(mx_ref, sm_ref, wa_ref, wb_ref, b_ref, o_ref):
    acc = lax.dot_general(mx_ref[...], wa_ref[...], (((1,), (1,)), ((), ())),
                          preferred_element_type=jnp.float32)
    acc = acc + lax.dot_general(sm_ref[...], wb_ref[...],
                                (((1,), (1,)), ((), ())),
                                preferred_element_type=jnp.float32)
    o_ref[...] = acc + b_ref[...]


def kernel(x, membership, W_merge, b_merge):
    n, d = x.shape
    bseg = NUM_SEGMENTS
    memb32 = membership.astype(jnp.int32)
    seg_max, seg_sum = _make_seg_reduce(n, d, bseg)(memb32, x)
    wa = W_merge[:, :d]
    wb = W_merge[:, d:]
    out = pl.pallas_call(
        _merge_body,
        out_shape=jax.ShapeDtypeStruct((bseg, d), jnp.float32),
    )(seg_max, seg_sum, wa, wb, b_merge.reshape(1, d))
    return out


# speedup vs baseline: 5.2757x; 1.1492x over previous
import functools

import jax
import jax.numpy as jnp
from jax import lax
from jax.experimental import pallas as pl
from jax.experimental.pallas import tpu as pltpu
from jax.experimental.pallas import tpu_sc as plsc

NUM_SEGMENTS = 512
NC = 2
NS = 16
L = 16
CR = 32

_NEG_INF = float("-inf")


@functools.lru_cache(maxsize=None)
def _make_seg_reduce(n, d, b):
    nw = NC * NS
    seg_per_w = b // nw
    n_sub = d // L
    assert n % 8 == 0 and n >= CR and d % L == 0 and b % nw == 0
    mesh = plsc.VectorSubcoreMesh(core_axis_name="c", subcore_axis_name="s",
                                  num_cores=NC, num_subcores=NS)

    @functools.partial(
        pl.kernel,
        out_type=(
            jax.ShapeDtypeStruct((b, d), jnp.float32),
            jax.ShapeDtypeStruct((b, d), jnp.float32),
        ),
        mesh=mesh,
        scratch_types=[
            pltpu.VMEM((n,), jnp.int32),
            pltpu.VMEM((8, d), jnp.float32),
            pltpu.VMEM((CR, d), jnp.float32),
            pltpu.VMEM((CR, d), jnp.float32),
            pltpu.VMEM((CR, d), jnp.float32),
            pltpu.VMEM((seg_per_w, d), jnp.float32),
            pltpu.VMEM((seg_per_w, d), jnp.float32),
            pltpu.SemaphoreType.DMA,
            pltpu.SemaphoreType.DMA,
            pltpu.SemaphoreType.DMA,
            pltpu.SemaphoreType.DMA,
        ],
        compiler_params=pltpu.CompilerParams(needs_layout_passes=False),
    )
    def seg_reduce(memb_hbm, x_hbm, max_hbm, sum_hbm,
                   memb_v, hbuf, fbuf0, fbuf1, tbuf, omax_v, osum_v,
                   sem_h, sem_f0, sem_f1, sem_t):
        wid = lax.axis_index("c") * NS + lax.axis_index("s")
        seg0 = wid * seg_per_w

        pltpu.sync_copy(memb_hbm, memb_v)

        targets = seg0 + lax.iota(jnp.int32, L)

        def lower_bound(tv):
            def step(_, carry):
                lo, hi = carry
                mid = lax.div(lo + hi, 2)
                vals = plsc.load_gather(memb_v, [jnp.minimum(mid, n - 1)])
                pred = vals < tv
                return jnp.where(pred, mid + 1, lo), jnp.where(pred, hi, mid)
            lo = jnp.zeros((L,), jnp.int32)
            hi = jnp.full((L,), n, jnp.int32)
            lo, hi = lax.fori_loop(0, 17, step, (lo, hi))
            return jnp.minimum(lo, n)

        starts = lower_bound(targets)
        ends = lower_bound(targets + 1)
        lanes = lax.iota(jnp.int32, L)

        def lane_extract(vec, idx):
            return jnp.sum(jnp.where(lanes == idx, vec, 0), axis=0)

        def accum_rows(s_idx, buf, nrows, valid_of_row):
            npart = 4 if nrows >= 8 else 2
            for c in range(n_sub):
                cs = pl.ds(c * L, L)
                pmax = [jnp.full((L,), _NEG_INF, jnp.float32)
                        for _ in range(npart)]
                psum = [jnp.zeros((L,), jnp.float32) for _ in range(npart)]
                for r in range(nrows):
                    v = buf[r, cs]
                    vmask = valid_of_row(r)
                    if vmask is None:
                        vm = vs = v
                    else:
                        vm = jnp.where(vmask, v, _NEG_INF)
                        vs = jnp.where(vmask, v, 0.0)
                    p = r % npart
                    pmax[p] = jnp.maximum(pmax[p], vm)
                    psum[p] = psum[p] + vs
                for p in range(1, npart):
                    pmax[0] = jnp.maximum(pmax[0], pmax[p])
                    psum[0] = psum[0] + psum[p]
                omax_v[s_idx, cs] = jnp.maximum(omax_v[s_idx, cs], pmax[0])
                osum_v[s_idx, cs] = osum_v[s_idx, cs] + psum[0]

        def do_segment(s_idx, _):
            lo = lane_extract(starts, s_idx)
            hi = lane_extract(ends, s_idx)
            lo8u = lax.div(lo + 7, 8) * 8
            nfull = lax.div(jnp.maximum(hi - lo8u, 0), CR)
            head_hi = jnp.minimum(lo8u, hi)
            head_needed = lo < head_hi
            hbase = jnp.minimum(jnp.maximum(lo8u - 8, 0), n - 8)
            t_lo = lo8u + nfull * CR
            tail_needed = t_lo < hi
            tbase = jnp.minimum(t_lo, n - CR)

            def start_chunk(k, buf, sem):
                base = pl.multiple_of(lo8u + k * CR, 8)
                pltpu.make_async_copy(
                    x_hbm.at[pl.ds(base, CR)], buf, sem).start()

            @pl.when(nfull > 0)
            def _():
                start_chunk(0, fbuf0, sem_f0)

            @pl.when(head_needed)
            def _():
                base = pl.multiple_of(hbase, 8)
                pltpu.make_async_copy(
                    x_hbm.at[pl.ds(base, 8)], hbuf, sem_h).start()

            @pl.when(tail_needed)
            def _():
                base = pl.multiple_of(tbase, 8)
                pltpu.make_async_copy(
                    x_hbm.at[pl.ds(base, CR)], tbuf, sem_t).start()

            ninf16 = jnp.full((L,), _NEG_INF, jnp.float32)
            zero16 = jnp.zeros((L,), jnp.float32)
            for c in range(n_sub):
                omax_v[s_idx, pl.ds(c * L, L)] = ninf16
                osum_v[s_idx, pl.ds(c * L, L)] = zero16

            def wait(buf, sem, rows):
                pltpu.make_async_copy(
                    x_hbm.at[pl.ds(0, rows)], buf, sem).wait()

            npairs = lax.div(nfull + 1, 2)

            def pair(i, _):
                k0 = 2 * i
                wait(fbuf0, sem_f0, CR)

                @pl.when(k0 + 1 < nfull)
                def _():
                    start_chunk(k0 + 1, fbuf1, sem_f1)

                accum_rows(s_idx, fbuf0, CR, lambda r: None)

                @pl.when(k0 + 1 < nfull)
                def _():
                    wait(fbuf1, sem_f1, CR)

                    @pl.when(k0 + 2 < nfull)
                    def _():
                        start_chunk(k0 + 2, fbuf0, sem_f0)

                    accum_rows(s_idx, fbuf1, CR, lambda r: None)

                return 0

            lax.fori_loop(0, npairs, pair, 0)

            @pl.when(head_needed)
            def _():
                wait(hbuf, sem_h, 8)
                accum_rows(
                    s_idx, hbuf, 8,
                    lambda r: jnp.logical_and(hbase + r >= lo,
                                              hbase + r < head_hi))

            @pl.when(tail_needed)
            def _():
                wait(tbuf, sem_t, CR)
                accum_rows(
                    s_idx, tbuf, CR,
                    lambda r: jnp.logical_and(tbase + r >= t_lo,
                                              tbase + r < hi))

            for c in range(n_sub):
                cs = pl.ds(c * L, L)
                m = omax_v[s_idx, cs]
                omax_v[s_idx, cs] = jnp.where(m == _NEG_INF, 0.0, m)
            return 0

        lax.fori_loop(0, seg_per_w, do_segment, 0)

        pltpu.sync_copy(omax_v, max_hbm.at[pl.ds(seg0, seg_per_w)])
        pltpu.sync_copy(osum_v, sum_hbm.at[pl.ds(seg0, seg_per_w)])

    return seg_reduce


def _merge_body(mx_ref, sm_ref, wa_ref, wb_ref, b_ref, o_ref):
    acc = lax.dot_general(mx_ref[...], wa_ref[...], (((1,), (1,)), ((), ())),
                          preferred_element_type=jnp.float32)
    acc = acc + lax.dot_general(sm_ref[...], wb_ref[...],
                                (((1,), (1,)), ((), ())),
                                preferred_element_type=jnp.float32)
    o_ref[...] = acc + b_ref[...]


def kernel(x, membership, W_merge, b_merge):
    n, d = x.shape
    bseg = NUM_SEGMENTS
    memb32 = membership.astype(jnp.int32)
    seg_max, seg_sum = _make_seg_reduce(n, d, bseg)(memb32, x)
    wa = W_merge[:, :d]
    wb = W_merge[:, d:]
    out = pl.pallas_call(
        _merge_body,
        out_shape=jax.ShapeDtypeStruct((bseg, d), jnp.float32),
    )(seg_max, seg_sum, wa, wb, b_merge.reshape(1, d))
    return out
